# baseline (speedup 1.0000x reference)
import functools

import jax
import jax.numpy as jnp
from jax.experimental import pallas as pl
from jax.experimental.pallas import tpu as pltpu
from jax.experimental.shard_map import shard_map
from jax.sharding import Mesh, PartitionSpec as P

_B = 16384
_A = 1000
_H = 16
_Q_INIT = 0.5
_FORGETTING = 0.05
_R = 1024


def _block_kernel(q_ref, idx_ref, rew_ref, rW1_ref, rb1_ref, rW2_ref, rb2_ref,
                  aW1_ref, ab1_ref, aW2_ref, ab2_ref,
                  qn_ref, ct_ref, lg_ref, pr_ref):
    idx = idx_ref[:, :1]
    rew = rew_ref[:, :1]
    q = q_ref[...]

    col = jax.lax.broadcasted_iota(jnp.int32, (_R, _A), 1)
    mask = col == idx

    q_decay = (1.0 - _FORGETTING) * q + _FORGETTING * _Q_INIT
    chosen_q = jnp.sum(jnp.where(mask, q, 0.0), axis=1, keepdims=True)

    h = jnp.tanh(chosen_q * rW1_ref[0, :][None, :]
                 + rew * rW1_ref[1, :][None, :]
                 + rb1_ref[0, :][None, :])
    chosen_new = jnp.sum(h * rW2_ref[0, :][None, :], axis=1, keepdims=True) \
        + rb2_ref[0, 0]

    q_new = jnp.where(mask, chosen_new, q_decay)

    hot = mask.astype(jnp.float32)
    g = jnp.dot(hot, aW1_ref[...], preferred_element_type=jnp.float32)
    h2 = jnp.tanh(g + ab1_ref[0, :][None, :])
    c_t = jnp.dot(h2, aW2_ref[...], preferred_element_type=jnp.float32) \
        + ab2_ref[0, :][None, :]

    logits = q_new + c_t
    m = jnp.max(logits, axis=1, keepdims=True)
    e = jnp.exp(logits - m)
    probs = e / jnp.sum(e, axis=1, keepdims=True)

    qn_ref[...] = q_new
    ct_ref[...] = c_t
    lg_ref[...] = logits
    pr_ref[...] = probs


def _run(q_prev, idx2, rew2, rW1, rb1, rW2, rb2, aW1, ab1, aW2, ab2,
         interpret=False):
    rows = q_prev.shape[0]
    nb = rows // _R
    row_spec = pl.BlockSpec((_R, _A), lambda i: (i, 0))
    tile_spec = pl.BlockSpec((_R, 128), lambda i: (i, 0))

    def full(shape):
        return pl.BlockSpec(shape, lambda i: (0,) * len(shape))

    out_shape = [jax.ShapeDtypeStruct((rows, _A), jnp.float32)] * 4
    return pl.pallas_call(
        _block_kernel,
        grid=(nb,),
        in_specs=[
            row_spec, tile_spec, tile_spec,
            full((2, _H)), full((1, _H)), full((1, _H)), full((1, 1)),
            full((_A, _H)), full((1, _H)), full((_H, _A)), full((1, _A)),
        ],
        out_specs=[row_spec] * 4,
        out_shape=out_shape,
        compiler_params=pltpu.CompilerParams(
            dimension_semantics=("arbitrary",)),
        interpret=interpret,
    )(q_prev, idx2, rew2, rW1, rb1, rW2, rb2, aW1, ab1, aW2, ab2)


def kernel(q_prev, prev_action_idx, prev_reward, rW1, rb1, rW2, rb2,
           aW1, ab1, aW2, ab2):
    idx2 = jnp.broadcast_to(
        prev_action_idx.astype(jnp.int32)[:, None], (_B, 128))
    rew2 = jnp.broadcast_to(prev_reward[:, None], (_B, 128))
    args = (q_prev, idx2, rew2,
            rW1, rb1.reshape(1, _H), rW2.reshape(1, _H), rb2.reshape(1, 1),
            aW1, ab1.reshape(1, _H), aW2, ab2.reshape(1, _A))

    devs = jax.devices()
    nd = 2 if len(devs) >= 2 else 1
    if nd > 1:
        mesh = Mesh(devs[:nd], ("x",))
        shard = P("x", None)
        repl = P(None, None)
        fn = shard_map(
            _run, mesh=mesh,
            in_specs=(shard, shard, shard,
                      repl, repl, repl, repl, repl, repl, repl, repl),
            out_specs=(shard,) * 4,
            check_rep=False)
    else:
        fn = _run
    q_new, c_t, logits, probs = fn(*args)
    return (q_new, c_t, logits, probs)


# speedup vs baseline: 2.0821x; 2.0821x over previous
import functools

import jax
import jax.numpy as jnp
from jax.experimental import pallas as pl
from jax.experimental.pallas import tpu as pltpu

_B = 16384
_A = 1000
_H = 16
_Q_INIT = 0.5
_FORGETTING = 0.05
_R = 1024


def _block_kernel(q_ref, idx_ref, rew_ref, rW1_ref, rb1_ref, rW2_ref, rb2_ref,
                  aW1t_ref, ab1_ref, aW2_ref, ab2_ref,
                  qn_ref, ct_ref, lg_ref, pr_ref):
    idx = idx_ref[...]
    rew = rew_ref[...]
    q = q_ref[...]

    col = jax.lax.broadcasted_iota(jnp.int32, (_R, _A), 1)
    mask = col == idx

    q_decay = (1.0 - _FORGETTING) * q + _FORGETTING * _Q_INIT
    chosen_q = jnp.sum(jnp.where(mask, q, 0.0), axis=1, keepdims=True)

    h = jnp.tanh(chosen_q * rW1_ref[0, :][None, :]
                 + rew * rW1_ref[1, :][None, :]
                 + rb1_ref[0, :][None, :])
    chosen_new = jnp.sum(h * rW2_ref[0, :][None, :], axis=1, keepdims=True) \
        + rb2_ref[0, 0]

    q_new = jnp.where(mask, chosen_new, q_decay)

    hot = mask.astype(jnp.float32)
    g = jax.lax.dot_general(hot, aW1t_ref[...],
                            (((1,), (1,)), ((), ())),
                            preferred_element_type=jnp.float32)
    h2 = jnp.tanh(g + ab1_ref[0, :][None, :])
    c_t = jnp.dot(h2, aW2_ref[...], preferred_element_type=jnp.float32) \
        + ab2_ref[0, :][None, :]

    logits = q_new + c_t
    m = jnp.max(logits, axis=1, keepdims=True)
    e = jnp.exp(logits - m)
    probs = e / jnp.sum(e, axis=1, keepdims=True)

    qn_ref[...] = q_new
    ct_ref[...] = c_t
    lg_ref[...] = logits
    pr_ref[...] = probs


@functools.partial(jax.jit, static_argnames=("interpret",))
def _run(q_prev, idx2, rew2, rW1, rb1, rW2, rb2, aW1t, ab1, aW2, ab2,
         interpret=False):
    nb = _B // _R
    row_spec = pl.BlockSpec((_R, _A), lambda i: (i, 0))
    vec_spec = pl.BlockSpec((_R, 1), lambda i: (i, 0))

    def full(shape):
        return pl.BlockSpec(shape, lambda i: (0,) * len(shape))

    out_shape = [jax.ShapeDtypeStruct((_B, _A), jnp.float32)] * 4
    return pl.pallas_call(
        _block_kernel,
        grid=(nb,),
        in_specs=[
            row_spec, vec_spec, vec_spec,
            full((2, _H)), full((1, _H)), full((1, _H)), full((1, 1)),
            full((_H, _A)), full((1, _H)), full((_H, _A)), full((1, _A)),
        ],
        out_specs=[row_spec] * 4,
        out_shape=out_shape,
        compiler_params=pltpu.CompilerParams(
            dimension_semantics=("arbitrary",)),
        interpret=interpret,
    )(q_prev, idx2, rew2, rW1, rb1, rW2, rb2, aW1t, ab1, aW2, ab2)


def kernel(q_prev, prev_action_idx, prev_reward, rW1, rb1, rW2, rb2,
           aW1, ab1, aW2, ab2):
    idx2 = prev_action_idx.astype(jnp.int32).reshape(_B, 1)
    rew2 = prev_reward.reshape(_B, 1)
    q_new, c_t, logits, probs = _run(
        q_prev, idx2, rew2,
        rW1, rb1.reshape(1, _H), rW2.reshape(1, _H), rb2.reshape(1, 1),
        aW1.T, ab1.reshape(1, _H), aW2, ab2.reshape(1, _A))
    return (q_new, c_t, logits, probs)


# speedup vs baseline: 2.1908x; 1.0522x over previous
import functools

import jax
import jax.numpy as jnp
from jax.experimental import pallas as pl
from jax.experimental.pallas import tpu as pltpu

_B = 16384
_A = 1000
_H = 16
_Q_INIT = 0.5
_FORGETTING = 0.05
_R = 1024


def _to_column(tile):
    t = tile.T
    return jnp.concatenate([t[:, s:s + 1] for s in range(8)], axis=0)


def _block_kernel(q_ref, idx_ref, rew_ref, rW1_ref, rb1_ref, rW2_ref, rb2_ref,
                  aW1t_ref, ab1_ref, aW2_ref, ab2_ref,
                  qn_ref, ct_ref, lg_ref, pr_ref):
    idx = _to_column(idx_ref[...])
    rew = _to_column(rew_ref[...])
    q = q_ref[...]

    col = jax.lax.broadcasted_iota(jnp.int32, (_R, _A), 1)
    mask = col == idx

    q_decay = (1.0 - _FORGETTING) * q + _FORGETTING * _Q_INIT
    chosen_q = jnp.sum(jnp.where(mask, q, 0.0), axis=1, keepdims=True)

    h = jnp.tanh(chosen_q * rW1_ref[0, :][None, :]
                 + rew * rW1_ref[1, :][None, :]
                 + rb1_ref[0, :][None, :])
    chosen_new = jnp.sum(h * rW2_ref[0, :][None, :], axis=1, keepdims=True) \
        + rb2_ref[0, 0]

    q_new = jnp.where(mask, chosen_new, q_decay)

    hot = mask.astype(jnp.float32)
    g = jax.lax.dot_general(hot, aW1t_ref[...],
                            (((1,), (1,)), ((), ())),
                            preferred_element_type=jnp.float32)
    h2 = jnp.tanh(g + ab1_ref[0, :][None, :])
    c_t = jnp.dot(h2, aW2_ref[...], preferred_element_type=jnp.float32) \
        + ab2_ref[0, :][None, :]

    logits = q_new + c_t
    m = jnp.max(logits, axis=1, keepdims=True)
    e = jnp.exp(logits - m)
    probs = e / jnp.sum(e, axis=1, keepdims=True)

    qn_ref[...] = q_new
    ct_ref[...] = c_t
    lg_ref[...] = logits
    pr_ref[...] = probs


@functools.partial(jax.jit, static_argnames=("interpret",))
def _run(q_prev, idx2, rew2, rW1, rb1, rW2, rb2, aW1t, ab1, aW2, ab2,
         interpret=False):
    nb = _B // _R
    row_spec = pl.BlockSpec((_R, _A), lambda i: (i, 0))
    vec_spec = pl.BlockSpec((_R // 128, 128), lambda i: (i, 0))

    def full(shape):
        return pl.BlockSpec(shape, lambda i: (0,) * len(shape))

    out_shape = [jax.ShapeDtypeStruct((_B, _A), jnp.float32)] * 4
    return pl.pallas_call(
        _block_kernel,
        grid=(nb,),
        in_specs=[
            row_spec, vec_spec, vec_spec,
            full((2, _H)), full((1, _H)), full((1, _H)), full((1, 1)),
            full((_H, _A)), full((1, _H)), full((_H, _A)), full((1, _A)),
        ],
        out_specs=[row_spec] * 4,
        out_shape=out_shape,
        compiler_params=pltpu.CompilerParams(
            dimension_semantics=("arbitrary",)),
        interpret=interpret,
    )(q_prev, idx2, rew2, rW1, rb1, rW2, rb2, aW1t, ab1, aW2, ab2)


def kernel(q_prev, prev_action_idx, prev_reward, rW1, rb1, rW2, rb2,
           aW1, ab1, aW2, ab2):
    idx2 = prev_action_idx.astype(jnp.int32).reshape(_B // 128, 128)
    rew2 = prev_reward.reshape(_B // 128, 128)
    q_new, c_t, logits, probs = _run(
        q_prev, idx2, rew2,
        rW1, rb1.reshape(1, _H), rW2.reshape(1, _H), rb2.reshape(1, 1),
        aW1.T, ab1.reshape(1, _H), aW2, ab2.reshape(1, _A))
    return (q_new, c_t, logits, probs)
